# baseline (speedup 1.0000x reference)
import jax
import jax.numpy as jnp
from jax import lax
from jax.experimental import pallas as pl
from jax.experimental.pallas import tpu as pltpu

TOKEN_ID = 31999
S_BLK = 512


def _fused_body(ids_col_ref, t_col_ref, wb_ref, emb_ref, out_ref,
                count_ref, tril_ref):
    j = pl.program_id(1)
    T = t_col_ref.shape[1]

    @pl.when((pl.program_id(0) == 0) & (j == 0))
    def _init_tril():
        i_col = lax.broadcasted_iota(jnp.int32, (S_BLK, S_BLK), 0)
        j_col = lax.broadcasted_iota(jnp.int32, (S_BLK, S_BLK), 1)
        tril_ref[...] = (j_col <= i_col).astype(jnp.float32)

    @pl.when(j == 0)
    def _reset_count():
        count_ref[0] = 0.0

    prefix = count_ref[0]

    mask_col = (ids_col_ref[0] == TOKEN_ID)
    mcol_f = mask_col.astype(jnp.float32)

    csum_col = jnp.dot(tril_ref[...], mcol_f,
                       preferred_element_type=jnp.float32)
    rank_col = csum_col + (prefix - 1.0)

    k_iota = lax.broadcasted_iota(jnp.int32, (S_BLK, T), 1).astype(jnp.float32)
    onehot = jnp.where((rank_col == k_iota) & mask_col, 1.0, 0.0)

    t_col = t_col_ref[0]
    t_scaled = (t_col - 1175.0) * (1.0 / 2350.0)
    vals_col = jnp.dot(onehot, t_scaled,
                       preferred_element_type=jnp.float32)

    vals_ones = jnp.concatenate(
        [vals_col, jnp.ones_like(vals_col)], axis=1)
    replace = jnp.dot(vals_ones, wb_ref[...],
                      preferred_element_type=jnp.float32)

    out_ref[0] = jnp.where(mask_col, replace, emb_ref[0])

    count_ref[0] = prefix + jnp.sum(mcol_f)


def kernel(inputs_embeds, input_ids, t_indices, W, b):
    B, S, H = inputs_embeds.shape
    T = t_indices.shape[0] // B
    ids_col3 = input_ids.reshape(B, S, 1)
    t_col3 = t_indices.reshape(B, T, 1)
    wb = jnp.concatenate([W, b.reshape(1, H)], axis=0)

    grid = (B, S // S_BLK)
    out = pl.pallas_call(
        _fused_body,
        grid=grid,
        in_specs=[
            pl.BlockSpec((1, S_BLK, 1), lambda i, j: (i, j, 0)),
            pl.BlockSpec((1, T, 1), lambda i, j: (i, 0, 0)),
            pl.BlockSpec((2, H), lambda i, j: (0, 0)),
            pl.BlockSpec((1, S_BLK, H), lambda i, j: (i, j, 0)),
        ],
        out_specs=pl.BlockSpec((1, S_BLK, H), lambda i, j: (i, j, 0)),
        out_shape=jax.ShapeDtypeStruct((B, S, H), inputs_embeds.dtype),
        scratch_shapes=[
            pltpu.SMEM((1,), jnp.float32),
            pltpu.VMEM((S_BLK, S_BLK), jnp.float32),
        ],
    )(ids_col3, t_col3, wb, inputs_embeds)
    return out


# speedup vs baseline: 1.0370x; 1.0370x over previous
import jax
import jax.numpy as jnp
from jax import lax
from jax.experimental import pallas as pl

TOKEN_ID = 31999
S_BLK = 1024


def _fused_body(ids_full_ref, ids_col_ref, t_col_ref, w_ref, bias_ref, emb_ref, out_ref):
    sb = pl.program_id(1) * S_BLK
    S = ids_full_ref.shape[2]
    T = t_col_ref.shape[1]

    ids_full = ids_full_ref[0]
    mask_full = (ids_full == TOKEN_ID)
    iota_s = lax.broadcasted_iota(jnp.int32, (1, S), 1)
    prefix = jnp.sum(
        jnp.where(mask_full & (iota_s < sb), 1.0, 0.0), dtype=jnp.float32
    )

    mask_col = (ids_col_ref[0] == TOKEN_ID)
    mcol_f = mask_col.astype(jnp.float32)

    i_col = lax.broadcasted_iota(jnp.int32, (S_BLK, S_BLK), 0)
    j_col = lax.broadcasted_iota(jnp.int32, (S_BLK, S_BLK), 1)
    tril = (j_col <= i_col).astype(jnp.float32)
    csum_col = jnp.dot(tril, mcol_f,
                       preferred_element_type=jnp.float32)

    rank_col = csum_col + (prefix - 1.0)

    k_iota = lax.broadcasted_iota(jnp.int32, (S_BLK, T), 1).astype(jnp.float32)
    onehot = jnp.where((rank_col == k_iota) & mask_col, 1.0, 0.0)

    t_col = t_col_ref[0]
    t_scaled = (t_col - 1175.0) * (1.0 / 2350.0)
    vals_col = jnp.dot(onehot, t_scaled,
                       preferred_element_type=jnp.float32)

    replace = vals_col * w_ref[...] + bias_ref[...]
    out_ref[0] = jnp.where(mask_col, replace, emb_ref[0])


def kernel(inputs_embeds, input_ids, t_indices, W, b):
    B, S, H = inputs_embeds.shape
    T = t_indices.shape[0] // B
    ids3 = input_ids.reshape(B, 1, S)
    ids_col3 = input_ids.reshape(B, S, 1)
    t_col3 = t_indices.reshape(B, T, 1)
    b2 = b.reshape(1, H)

    grid = (B, S // S_BLK)
    out = pl.pallas_call(
        _fused_body,
        grid=grid,
        in_specs=[
            pl.BlockSpec((1, 1, S), lambda i, j: (i, 0, 0)),
            pl.BlockSpec((1, S_BLK, 1), lambda i, j: (i, j, 0)),
            pl.BlockSpec((1, T, 1), lambda i, j: (i, 0, 0)),
            pl.BlockSpec((1, H), lambda i, j: (0, 0)),
            pl.BlockSpec((1, H), lambda i, j: (0, 0)),
            pl.BlockSpec((1, S_BLK, H), lambda i, j: (i, j, 0)),
        ],
        out_specs=pl.BlockSpec((1, S_BLK, H), lambda i, j: (i, j, 0)),
        out_shape=jax.ShapeDtypeStruct((B, S, H), inputs_embeds.dtype),
    )(ids3, ids_col3, t_col3, W, b2, inputs_embeds)
    return out
